# baseline (speedup 1.0000x reference)
import functools

import jax
import jax.numpy as jnp
from jax.experimental import pallas as pl
from jax.experimental.pallas import tpu as pltpu

_VMEM_LIMIT = 48 * 1024 * 1024


def _leaky(v, slope):
    return jnp.where(v >= 0.0, v, slope * v)


def _fused_kernel(x_ref, w1_ref, b1_ref, w2_ref, b2_ref, wi_ref, bi_ref,
                  wd_ref, out_ref, down_ref, xp_ref, h1_ref, op_ref,
                  *, H, W, slope):
    cin = x_ref.shape[1]
    cout = out_ref.shape[1]
    Ho, Wo = H // 2, W // 2

    xhwc = jnp.transpose(x_ref[0].astype(jnp.bfloat16), (1, 2, 0))
    xp_ref[...] = jnp.zeros((H + 2, W + 2, cin), jnp.bfloat16)
    xp_ref[1:H + 1, 1:W + 1, :] = xhwc

    acc = jnp.zeros((H * W, cout), jnp.float32)
    for dy in range(3):
        for dx in range(3):
            patch = xp_ref[dy:dy + H, dx:dx + W, :].reshape(H * W, cin)
            acc = acc + jnp.dot(patch, w1_ref[dy * 3 + dx],
                                preferred_element_type=jnp.float32)
    h1 = _leaky(acc + b1_ref[...], slope).astype(jnp.bfloat16)

    h1_ref[...] = jnp.zeros((H + 2, W + 2, cout), jnp.bfloat16)
    h1_ref[1:H + 1, 1:W + 1, :] = h1.reshape(H, W, cout)

    acc2 = jnp.zeros((H * W, cout), jnp.float32)
    for dy in range(3):
        for dx in range(3):
            patch = h1_ref[dy:dy + H, dx:dx + W, :].reshape(H * W, cout)
            acc2 = acc2 + jnp.dot(patch, w2_ref[dy * 3 + dx],
                                  preferred_element_type=jnp.float32)
    res = _leaky(acc2 + b2_ref[...], slope)
    xin = xp_ref[1:H + 1, 1:W + 1, :].reshape(H * W, cin)
    ident = jnp.dot(xin, wi_ref[...],
                    preferred_element_type=jnp.float32) + bi_ref[...]
    outv = res + 2.0 * ident
    out_ref[0] = jnp.transpose(outv.reshape(H, W, cout), (2, 0, 1))

    op_ref[...] = jnp.zeros((H + 2, W + 2, cout), jnp.bfloat16)
    op_ref[1:H + 1, 1:W + 1, :] = outv.astype(jnp.bfloat16).reshape(H, W, cout)

    acc3 = jnp.zeros((Ho * Wo, cout), jnp.float32)
    for ky in range(4):
        p = ky % 2
        s = ky - p
        rows = op_ref[s:s + H, :, :].reshape(Ho, 2, W + 2, cout)[:, p]
        for kx in range(4):
            q = kx % 2
            t = kx - q
            cols = rows[:, t:t + W, :].reshape(Ho, Wo, 2, cout)[:, :, q]
            acc3 = acc3 + jnp.dot(cols.reshape(Ho * Wo, cout),
                                  wd_ref[ky * 4 + kx],
                                  preferred_element_type=jnp.float32)
    down_ref[0] = jnp.transpose(acc3.reshape(Ho, Wo, cout), (2, 0, 1))


def kernel(x, res_w1, res_b1, res_w2, res_b2, id_w, id_b, down_w):
    N, cin, H, W = x.shape
    cout = res_w1.shape[0]
    Ho, Wo = H // 2, W // 2
    slope = 0.2

    def conv_w(w):
        co, ci = w.shape[0], w.shape[1]
        k = w.shape[2] * w.shape[3]
        return jnp.transpose(w, (2, 3, 1, 0)).reshape(k, ci, co).astype(jnp.bfloat16)

    w1 = conv_w(res_w1)
    w2 = conv_w(res_w2)
    wd = conv_w(down_w)
    wi = id_w.reshape(cout, cin).T.astype(jnp.bfloat16)
    b1 = res_b1.reshape(1, cout).astype(jnp.float32)
    b2 = res_b2.reshape(1, cout).astype(jnp.float32)
    bi = id_b.reshape(1, cout).astype(jnp.float32)

    out, down = pl.pallas_call(
        functools.partial(_fused_kernel, H=H, W=W, slope=slope),
        out_shape=[
            jax.ShapeDtypeStruct((N, cout, H, W), jnp.float32),
            jax.ShapeDtypeStruct((N, cout, Ho, Wo), jnp.float32),
        ],
        grid_spec=pltpu.PrefetchScalarGridSpec(
            num_scalar_prefetch=0,
            grid=(N,),
            in_specs=[
                pl.BlockSpec((1, cin, H, W), lambda n: (n, 0, 0, 0)),
                pl.BlockSpec((9, cin, cout), lambda n: (0, 0, 0)),
                pl.BlockSpec((1, cout), lambda n: (0, 0)),
                pl.BlockSpec((9, cout, cout), lambda n: (0, 0, 0)),
                pl.BlockSpec((1, cout), lambda n: (0, 0)),
                pl.BlockSpec((cin, cout), lambda n: (0, 0)),
                pl.BlockSpec((1, cout), lambda n: (0, 0)),
                pl.BlockSpec((16, cout, cout), lambda n: (0, 0, 0)),
            ],
            out_specs=[
                pl.BlockSpec((1, cout, H, W), lambda n: (n, 0, 0, 0)),
                pl.BlockSpec((1, cout, Ho, Wo), lambda n: (n, 0, 0, 0)),
            ],
            scratch_shapes=[
                pltpu.VMEM((H + 2, W + 2, cin), jnp.bfloat16),
                pltpu.VMEM((H + 2, W + 2, cout), jnp.bfloat16),
                pltpu.VMEM((H + 2, W + 2, cout), jnp.bfloat16),
            ],
        ),
        compiler_params=pltpu.CompilerParams(
            dimension_semantics=("parallel",), vmem_limit_bytes=_VMEM_LIMIT),
    )(x, w1, b1, w2, b2, wi, bi, wd)

    return down, out


# speedup vs baseline: 1.1257x; 1.1257x over previous
import functools

import jax
import jax.numpy as jnp
from jax.experimental import pallas as pl
from jax.experimental.pallas import tpu as pltpu

_VMEM_LIMIT = 48 * 1024 * 1024


def _leaky(v, slope):
    return jnp.where(v >= 0.0, v, slope * v)


def _fused_kernel(x_ref, w1_ref, b1_ref, w2_ref, b2_ref, wi_ref, bi_ref,
                  wd_ref, out_ref, down_ref, xp_ref, h1_ref, op_ref,
                  *, H, W, slope):
    cin = x_ref.shape[1]
    cout = out_ref.shape[1]
    Ho, Wo = H // 2, W // 2

    xhwc = jnp.transpose(x_ref[0].astype(jnp.bfloat16), (1, 0))
    xp_ref[...] = jnp.zeros((H + 2, W + 2, cin), jnp.bfloat16)
    xp_ref[1:H + 1, 1:W + 1, :] = xhwc.reshape(H, W, cin)

    acc = jnp.zeros((H * W, cout), jnp.float32)
    for dy in range(3):
        for dx in range(3):
            patch = xp_ref[dy:dy + H, dx:dx + W, :].reshape(H * W, cin)
            acc = acc + jnp.dot(patch, w1_ref[dy * 3 + dx],
                                preferred_element_type=jnp.float32)
    h1 = _leaky(acc + b1_ref[...], slope).astype(jnp.bfloat16)

    h1_ref[...] = jnp.zeros((H + 2, W + 2, cout), jnp.bfloat16)
    h1_ref[1:H + 1, 1:W + 1, :] = h1.reshape(H, W, cout)

    acc2 = jnp.zeros((H * W, cout), jnp.float32)
    for dy in range(3):
        for dx in range(3):
            patch = h1_ref[dy:dy + H, dx:dx + W, :].reshape(H * W, cout)
            acc2 = acc2 + jnp.dot(patch, w2_ref[dy * 3 + dx],
                                  preferred_element_type=jnp.float32)
    res = _leaky(acc2 + b2_ref[...], slope)
    xin = xp_ref[1:H + 1, 1:W + 1, :].reshape(H * W, cin)
    ident = jnp.dot(xin, wi_ref[...],
                    preferred_element_type=jnp.float32) + bi_ref[...]
    outv = res + 2.0 * ident
    out_ref[0] = jnp.transpose(outv, (1, 0))

    op_ref[...] = jnp.zeros((H + 2, W + 2, cout), jnp.bfloat16)
    op_ref[1:H + 1, 1:W + 1, :] = outv.astype(jnp.bfloat16).reshape(H, W, cout)

    acc3 = jnp.zeros((Ho * Wo, cout), jnp.float32)
    for ky in range(4):
        p = ky % 2
        s = ky - p
        rows = op_ref[s:s + H, :, :].reshape(Ho, 2, W + 2, cout)[:, p]
        for kx in range(4):
            q = kx % 2
            t = kx - q
            cols = rows[:, t:t + W, :].reshape(Ho, Wo, 2, cout)[:, :, q]
            acc3 = acc3 + jnp.dot(cols.reshape(Ho * Wo, cout),
                                  wd_ref[ky * 4 + kx],
                                  preferred_element_type=jnp.float32)
    down_ref[0] = jnp.transpose(acc3, (1, 0))


def kernel(x, res_w1, res_b1, res_w2, res_b2, id_w, id_b, down_w):
    N, cin, H, W = x.shape
    cout = res_w1.shape[0]
    Ho, Wo = H // 2, W // 2
    slope = 0.2

    def conv_w(w):
        co, ci = w.shape[0], w.shape[1]
        k = w.shape[2] * w.shape[3]
        return jnp.transpose(w, (2, 3, 1, 0)).reshape(k, ci, co).astype(jnp.bfloat16)

    w1 = conv_w(res_w1)
    w2 = conv_w(res_w2)
    wd = conv_w(down_w)
    wi = id_w.reshape(cout, cin).T.astype(jnp.bfloat16)
    b1 = res_b1.reshape(1, cout).astype(jnp.float32)
    b2 = res_b2.reshape(1, cout).astype(jnp.float32)
    bi = id_b.reshape(1, cout).astype(jnp.float32)

    out, down = pl.pallas_call(
        functools.partial(_fused_kernel, H=H, W=W, slope=slope),
        out_shape=[
            jax.ShapeDtypeStruct((N, cout, H * W), jnp.float32),
            jax.ShapeDtypeStruct((N, cout, Ho * Wo), jnp.float32),
        ],
        grid_spec=pltpu.PrefetchScalarGridSpec(
            num_scalar_prefetch=0,
            grid=(N,),
            in_specs=[
                pl.BlockSpec((1, cin, H * W), lambda n: (n, 0, 0)),
                pl.BlockSpec((9, cin, cout), lambda n: (0, 0, 0)),
                pl.BlockSpec((1, cout), lambda n: (0, 0)),
                pl.BlockSpec((9, cout, cout), lambda n: (0, 0, 0)),
                pl.BlockSpec((1, cout), lambda n: (0, 0)),
                pl.BlockSpec((cin, cout), lambda n: (0, 0)),
                pl.BlockSpec((1, cout), lambda n: (0, 0)),
                pl.BlockSpec((16, cout, cout), lambda n: (0, 0, 0)),
            ],
            out_specs=[
                pl.BlockSpec((1, cout, H * W), lambda n: (n, 0, 0)),
                pl.BlockSpec((1, cout, Ho * Wo), lambda n: (n, 0, 0)),
            ],
            scratch_shapes=[
                pltpu.VMEM((H + 2, W + 2, cin), jnp.bfloat16),
                pltpu.VMEM((H + 2, W + 2, cout), jnp.bfloat16),
                pltpu.VMEM((H + 2, W + 2, cout), jnp.bfloat16),
            ],
        ),
        compiler_params=pltpu.CompilerParams(
            dimension_semantics=("parallel",), vmem_limit_bytes=_VMEM_LIMIT),
    )(x.reshape(N, cin, H * W), w1, b1, w2, b2, wi, bi, wd)

    return down.reshape(N, cout, Ho, Wo), out.reshape(N, cout, H, W)


# speedup vs baseline: 1.2972x; 1.1523x over previous
import functools

import jax
import jax.numpy as jnp
from jax.experimental import pallas as pl
from jax.experimental.pallas import tpu as pltpu

_VMEM_LIMIT = 48 * 1024 * 1024


def _leaky(v, slope):
    return jnp.where(v >= 0.0, v, slope * v)


def _fused_kernel(x_ref, w1_ref, b1_ref, w2_ref, b2_ref, wi_ref, bi_ref,
                  wd_ref, out_ref, down_ref, *, H, W, slope):
    cin = x_ref.shape[1]
    cout = out_ref.shape[1]
    Ho, Wo = H // 2, W // 2
    Wp = W + 8
    P = H * Wp
    L1 = (H + 2) * Wp
    T = Wp + 1

    bf = jnp.bfloat16
    f32 = jnp.float32

    xt = jnp.transpose(x_ref[0].astype(bf), (1, 0)).reshape(H, W, cin)
    xrow = jnp.concatenate(
        [jnp.zeros((H, 1, cin), bf), xt, jnp.zeros((H, Wp - W - 1, cin), bf)],
        axis=1).reshape(H * Wp, cin)
    xf = jnp.concatenate(
        [jnp.zeros((Wp, cin), bf), xrow, jnp.zeros((2 * Wp, cin), bf)], axis=0)

    g1 = jnp.concatenate([xf[0:L1], xf[1:L1 + 1], xf[2:L1 + 2]], axis=1)

    acc1 = jnp.zeros((P, cout), f32)
    for dy in range(3):
        acc1 = acc1 + jnp.dot(g1[dy * Wp: dy * Wp + P], w1_ref[dy],
                              preferred_element_type=f32)
    h1 = _leaky(acc1 + b1_ref[...], slope)

    colid = jax.lax.broadcasted_iota(jnp.int32, (P, 1), 0) % Wp
    mask = colid < W
    h1m = jnp.where(mask, h1, 0.0).astype(bf)

    h1s = jnp.concatenate(
        [jnp.zeros((T, cout), bf), h1m,
         jnp.zeros((L1 + 2 - T - P, cout), bf)], axis=0)
    g2 = jnp.concatenate([h1s[0:L1], h1s[1:L1 + 1], h1s[2:L1 + 2]], axis=1)

    acc2 = jnp.zeros((P, cout), f32)
    for ey in range(3):
        acc2 = acc2 + jnp.dot(g2[ey * Wp: ey * Wp + P], w2_ref[ey],
                              preferred_element_type=f32)
    res = _leaky(acc2 + b2_ref[...], slope)
    ident = jnp.dot(g1[Wp: Wp + P], wi_ref[...],
                    preferred_element_type=f32) + bi_ref[...]
    outf = res + 2.0 * ident

    out3 = outf.reshape(H, Wp, cout)[:, :W, :].reshape(H * W, cout)
    out_ref[0] = jnp.transpose(out3, (1, 0))

    Wd = Wp + 8
    outm = jnp.where(mask, outf, 0.0).astype(bf).reshape(H, Wp, cout)
    orow = jnp.concatenate(
        [jnp.zeros((H, 1, cout), bf), outm,
         jnp.zeros((H, Wd - Wp - 1, cout), bf)], axis=1).reshape(H * Wd, cout)
    osv = jnp.concatenate(
        [jnp.zeros((Wd, cout), bf), orow, jnp.zeros((3 * Wd, cout), bf)],
        axis=0).reshape(H + 4, Wd, cout)

    Ld = Ho * Wd
    accd = jnp.zeros((Ld, cout), f32)
    for ky in range(4):
        p2 = ky % 2
        b = ky - p2
        dec = osv[b: b + H + 2].reshape((H + 2) // 2, 2, Wd, cout)[:, p2]
        fk = dec.reshape((H + 2) // 2 * Wd, cout)
        g = jnp.concatenate(
            [fk[0:Ld], fk[1:Ld + 1], fk[2:Ld + 2], fk[3:Ld + 3]], axis=1)
        accd = accd + jnp.dot(g, wd_ref[ky], preferred_element_type=f32)

    dn = accd.reshape(Ho, Wd, cout)[:, 0:2 * Wo, :]
    dn = dn.reshape(Ho, Wo, 2, cout)[:, :, 0].reshape(Ho * Wo, cout)
    down_ref[0] = jnp.transpose(dn, (1, 0))


def kernel(x, res_w1, res_b1, res_w2, res_b2, id_w, id_b, down_w):
    N, cin, H, W = x.shape
    cout = res_w1.shape[0]
    Ho, Wo = H // 2, W // 2
    slope = 0.2

    def conv_w(w, kh, kw):
        co, ci = w.shape[0], w.shape[1]
        return jnp.transpose(w, (2, 3, 1, 0)).reshape(
            kh, kw * ci, co).astype(jnp.bfloat16)

    w1 = conv_w(res_w1, 3, 3)
    w2 = conv_w(res_w2, 3, 3)
    wd = conv_w(down_w, 4, 4)
    wi0 = id_w.reshape(cout, cin).T.astype(jnp.bfloat16)
    wi = jnp.concatenate(
        [jnp.zeros((cin, cout), jnp.bfloat16), wi0,
         jnp.zeros((cin, cout), jnp.bfloat16)], axis=0)
    b1 = res_b1.reshape(1, cout).astype(jnp.float32)
    b2 = res_b2.reshape(1, cout).astype(jnp.float32)
    bi = id_b.reshape(1, cout).astype(jnp.float32)

    out, down = pl.pallas_call(
        functools.partial(_fused_kernel, H=H, W=W, slope=slope),
        out_shape=[
            jax.ShapeDtypeStruct((N, cout, H * W), jnp.float32),
            jax.ShapeDtypeStruct((N, cout, Ho * Wo), jnp.float32),
        ],
        grid_spec=pltpu.PrefetchScalarGridSpec(
            num_scalar_prefetch=0,
            grid=(N,),
            in_specs=[
                pl.BlockSpec((1, cin, H * W), lambda n: (n, 0, 0)),
                pl.BlockSpec((3, 3 * cin, cout), lambda n: (0, 0, 0)),
                pl.BlockSpec((1, cout), lambda n: (0, 0)),
                pl.BlockSpec((3, 3 * cout, cout), lambda n: (0, 0, 0)),
                pl.BlockSpec((1, cout), lambda n: (0, 0)),
                pl.BlockSpec((3 * cin, cout), lambda n: (0, 0)),
                pl.BlockSpec((1, cout), lambda n: (0, 0)),
                pl.BlockSpec((4, 4 * cout, cout), lambda n: (0, 0, 0)),
            ],
            out_specs=[
                pl.BlockSpec((1, cout, H * W), lambda n: (n, 0, 0)),
                pl.BlockSpec((1, cout, Ho * Wo), lambda n: (n, 0, 0)),
            ],
        ),
        compiler_params=pltpu.CompilerParams(
            dimension_semantics=("parallel",), vmem_limit_bytes=_VMEM_LIMIT),
    )(x.reshape(N, cin, H * W), w1, b1, w2, b2, wi, bi, wd)

    return down.reshape(N, cout, Ho, Wo), out.reshape(N, cout, H, W)


# speedup vs baseline: 1.5423x; 1.1889x over previous
import functools

import jax
import jax.numpy as jnp
from jax.experimental import pallas as pl
from jax.experimental.pallas import tpu as pltpu

_VMEM_LIMIT = 48 * 1024 * 1024


def _leaky(v, slope):
    return jnp.where(v >= 0.0, v, slope * v)


def _fused_kernel(x_ref, w1_ref, b1_ref, w2_ref, b2_ref, wi_ref, bi_ref,
                  wde_ref, wdo_ref, m_ref, out_ref, down_ref, *, H, W, slope):
    cin = x_ref.shape[1]
    cout = out_ref.shape[1]
    Ho, Wo = H // 2, W // 2
    Wp = W + 8
    P = H * Wp
    L1 = (H + 2) * Wp
    T = Wp + 1

    bf = jnp.bfloat16
    f32 = jnp.float32

    xt = jnp.transpose(x_ref[0].astype(bf), (1, 0)).reshape(H, W, cin)
    xrow = jnp.concatenate(
        [jnp.zeros((H, 1, cin), bf), xt, jnp.zeros((H, Wp - W - 1, cin), bf)],
        axis=1).reshape(H * Wp, cin)
    xf = jnp.concatenate(
        [jnp.zeros((Wp, cin), bf), xrow, jnp.zeros((2 * Wp, cin), bf)], axis=0)

    g1 = jnp.concatenate([xf[0:L1], xf[1:L1 + 1], xf[2:L1 + 2]], axis=1)

    acc1 = jnp.zeros((P, cout), f32)
    for dy in range(3):
        acc1 = acc1 + jnp.dot(g1[dy * Wp: dy * Wp + P], w1_ref[dy],
                              preferred_element_type=f32)
    h1 = _leaky(acc1 + b1_ref[...], slope)

    h1m = (h1 * m_ref[...]).astype(bf)

    h1s = jnp.concatenate(
        [jnp.zeros((T, cout), bf), h1m,
         jnp.zeros((L1 + 2 - T - P, cout), bf)], axis=0)
    g2 = jnp.concatenate([h1s[0:L1], h1s[1:L1 + 1], h1s[2:L1 + 2]], axis=1)

    acc2 = jnp.zeros((P, cout), f32)
    for ey in range(3):
        acc2 = acc2 + jnp.dot(g2[ey * Wp: ey * Wp + P], w2_ref[ey],
                              preferred_element_type=f32)
    res = _leaky(acc2 + b2_ref[...], slope)
    ident = jnp.dot(g1[Wp: Wp + P], wi_ref[...],
                    preferred_element_type=f32) + bi_ref[...]
    outf = res + 2.0 * ident

    out3 = outf.reshape(H, Wp, cout)[:, :W, :].reshape(H * W, cout)
    out_ref[0] = jnp.transpose(out3, (1, 0))

    Wd = Wo + 8
    outm = (outf * m_ref[...]).astype(bf).reshape(H, Wp // 2, 2, cout)
    ev = jnp.concatenate(
        [outm[:, :, 0], jnp.zeros((H, Wd - Wp // 2, cout), bf)],
        axis=1).reshape(H * Wd, cout)
    od = jnp.concatenate(
        [jnp.zeros((H, 1, cout), bf), outm[:, :, 1],
         jnp.zeros((H, Wd - Wp // 2 - 1, cout), bf)],
        axis=1).reshape(H * Wd, cout)
    zr = jnp.zeros((Wd, cout), bf)
    z3 = jnp.zeros((3 * Wd, cout), bf)
    ep = jnp.concatenate([zr, ev, z3], axis=0).reshape(H + 4, Wd, cout)
    op = jnp.concatenate([zr, od, z3], axis=0).reshape(H + 4, Wd, cout)

    Ld = Ho * Wd
    accd = jnp.zeros((Ld, cout), f32)
    for ky in range(4):
        p2 = ky % 2
        b = ky - p2
        for src, w_ref in ((ep, wde_ref), (op, wdo_ref)):
            dec = src[b: b + H + 2].reshape((H + 2) // 2, 2, Wd, cout)[:, p2]
            fk = dec.reshape((H + 2) // 2 * Wd, cout)
            g = jnp.concatenate([fk[0:Ld], fk[1:Ld + 1]], axis=1)
            accd = accd + jnp.dot(g, w_ref[ky], preferred_element_type=f32)

    dn = accd.reshape(Ho, Wd, cout)[:, 0:Wo, :].reshape(Ho * Wo, cout)
    down_ref[0] = jnp.transpose(dn, (1, 0))


def kernel(x, res_w1, res_b1, res_w2, res_b2, id_w, id_b, down_w):
    N, cin, H, W = x.shape
    cout = res_w1.shape[0]
    Ho, Wo = H // 2, W // 2
    slope = 0.2

    def conv_w(w, kh, kw):
        co, ci = w.shape[0], w.shape[1]
        return jnp.transpose(w, (2, 3, 1, 0)).reshape(
            kh, kw * ci, co).astype(jnp.bfloat16)

    w1 = conv_w(res_w1, 3, 3)
    w2 = conv_w(res_w2, 3, 3)
    wd4 = jnp.transpose(down_w, (2, 3, 1, 0)).astype(jnp.bfloat16)
    wde = wd4[:, 1::2].reshape(4, 2 * cout, cout)
    wdo = wd4[:, 0::2].reshape(4, 2 * cout, cout)
    wi0 = id_w.reshape(cout, cin).T.astype(jnp.bfloat16)
    wi = jnp.concatenate(
        [jnp.zeros((cin, cout), jnp.bfloat16), wi0,
         jnp.zeros((cin, cout), jnp.bfloat16)], axis=0)
    b1 = res_b1.reshape(1, cout).astype(jnp.float32)
    b2 = res_b2.reshape(1, cout).astype(jnp.float32)
    bi = id_b.reshape(1, cout).astype(jnp.float32)
    Wp = W + 8
    m = jnp.tile(
        (jnp.arange(Wp) < W).astype(jnp.float32), H).reshape(H * Wp, 1)

    out, down = pl.pallas_call(
        functools.partial(_fused_kernel, H=H, W=W, slope=slope),
        out_shape=[
            jax.ShapeDtypeStruct((N, cout, H * W), jnp.float32),
            jax.ShapeDtypeStruct((N, cout, Ho * Wo), jnp.float32),
        ],
        grid_spec=pltpu.PrefetchScalarGridSpec(
            num_scalar_prefetch=0,
            grid=(N,),
            in_specs=[
                pl.BlockSpec((1, cin, H * W), lambda n: (n, 0, 0)),
                pl.BlockSpec((3, 3 * cin, cout), lambda n: (0, 0, 0)),
                pl.BlockSpec((1, cout), lambda n: (0, 0)),
                pl.BlockSpec((3, 3 * cout, cout), lambda n: (0, 0, 0)),
                pl.BlockSpec((1, cout), lambda n: (0, 0)),
                pl.BlockSpec((3 * cin, cout), lambda n: (0, 0)),
                pl.BlockSpec((1, cout), lambda n: (0, 0)),
                pl.BlockSpec((4, 2 * cout, cout), lambda n: (0, 0, 0)),
                pl.BlockSpec((4, 2 * cout, cout), lambda n: (0, 0, 0)),
                pl.BlockSpec((H * Wp, 1), lambda n: (0, 0)),
            ],
            out_specs=[
                pl.BlockSpec((1, cout, H * W), lambda n: (n, 0, 0)),
                pl.BlockSpec((1, cout, Ho * Wo), lambda n: (n, 0, 0)),
            ],
        ),
        compiler_params=pltpu.CompilerParams(
            dimension_semantics=("parallel",), vmem_limit_bytes=_VMEM_LIMIT),
    )(x.reshape(N, cin, H * W), w1, b1, w2, b2, wi, bi, wde, wdo, m)

    return down.reshape(N, cout, Ho, Wo), out.reshape(N, cout, H, W)


# speedup vs baseline: 1.5804x; 1.0247x over previous
import functools

import jax
import jax.numpy as jnp
from jax.experimental import pallas as pl
from jax.experimental.pallas import tpu as pltpu

_VMEM_LIMIT = 48 * 1024 * 1024


def _leaky(v, slope):
    return jnp.where(v >= 0.0, v, slope * v)


def _fused_kernel(x_ref, w1_ref, b1_ref, w2_ref, b2_ref, wi_ref, bi_ref,
                  wde_ref, wdo_ref, out_ref, down_ref, *, H, W, slope):
    cin = x_ref.shape[1]
    cout = out_ref.shape[1]
    Ho, Wo = H // 2, W // 2
    Wp = W + 8
    P = H * Wp
    L1 = (H + 2) * Wp
    T = Wp + 1

    bf = jnp.bfloat16
    f32 = jnp.float32

    xt = jnp.transpose(x_ref[0].astype(bf), (1, 0)).reshape(H, W, cin)
    xrow = jnp.concatenate(
        [jnp.zeros((H, 1, cin), bf), xt, jnp.zeros((H, Wp - W - 1, cin), bf)],
        axis=1).reshape(H * Wp, cin)
    xf = jnp.concatenate(
        [jnp.zeros((Wp, cin), bf), xrow, jnp.zeros((2 * Wp, cin), bf)], axis=0)

    g1 = jnp.concatenate([xf[0:L1], xf[1:L1 + 1], xf[2:L1 + 2]], axis=1)

    acc1 = jnp.zeros((P, cout), f32)
    for dy in range(3):
        acc1 = acc1 + jnp.dot(g1[dy * Wp: dy * Wp + P], w1_ref[dy],
                              preferred_element_type=f32)
    h1 = _leaky(acc1 + b1_ref[...], slope)

    h1b = h1.astype(bf).reshape(H, Wp, cout)
    h1m = jnp.concatenate(
        [h1b[:, :W, :], jnp.zeros((H, Wp - W, cout), bf)],
        axis=1).reshape(P, cout)

    h1s = jnp.concatenate(
        [jnp.zeros((T, cout), bf), h1m,
         jnp.zeros((L1 + 2 - T - P, cout), bf)], axis=0)
    g2 = jnp.concatenate([h1s[0:L1], h1s[1:L1 + 1], h1s[2:L1 + 2]], axis=1)

    acc2 = jnp.zeros((P, cout), f32)
    for ey in range(3):
        acc2 = acc2 + jnp.dot(g2[ey * Wp: ey * Wp + P], w2_ref[ey],
                              preferred_element_type=f32)
    res = _leaky(acc2 + b2_ref[...], slope)
    ident = jnp.dot(g1[Wp: Wp + P], wi_ref[...],
                    preferred_element_type=f32) + bi_ref[...]
    outf = res + 2.0 * ident

    out3 = outf.reshape(H, Wp, cout)[:, :W, :].reshape(H * W, cout)
    out_ref[0] = jnp.transpose(out3, (1, 0))

    Wd = Wo + 8
    outm = out3.astype(bf).reshape(H, Wo, 2, cout)
    ev = jnp.concatenate(
        [outm[:, :, 0], jnp.zeros((H, Wd - Wo, cout), bf)],
        axis=1).reshape(H * Wd, cout)
    od = jnp.concatenate(
        [jnp.zeros((H, 1, cout), bf), outm[:, :, 1],
         jnp.zeros((H, Wd - Wo - 1, cout), bf)],
        axis=1).reshape(H * Wd, cout)
    zr = jnp.zeros((Wd, cout), bf)
    z3 = jnp.zeros((3 * Wd, cout), bf)
    ep = jnp.concatenate([zr, ev, z3], axis=0).reshape(H + 4, Wd, cout)
    op = jnp.concatenate([zr, od, z3], axis=0).reshape(H + 4, Wd, cout)

    Ld = Ho * Wd
    accd = jnp.zeros((Ld, cout), f32)
    for ky in range(4):
        p2 = ky % 2
        b = ky - p2
        for src, w_ref in ((ep, wde_ref), (op, wdo_ref)):
            dec = src[b: b + H + 2].reshape((H + 2) // 2, 2, Wd, cout)[:, p2]
            fk = dec.reshape((H + 2) // 2 * Wd, cout)
            g = jnp.concatenate([fk[0:Ld], fk[1:Ld + 1]], axis=1)
            accd = accd + jnp.dot(g, w_ref[ky], preferred_element_type=f32)

    dn = accd.reshape(Ho, Wd, cout)[:, 0:Wo, :].reshape(Ho * Wo, cout)
    down_ref[0] = jnp.transpose(dn, (1, 0))


def kernel(x, res_w1, res_b1, res_w2, res_b2, id_w, id_b, down_w):
    N, cin, H, W = x.shape
    cout = res_w1.shape[0]
    Ho, Wo = H // 2, W // 2
    slope = 0.2

    def conv_w(w, kh, kw):
        co, ci = w.shape[0], w.shape[1]
        return jnp.transpose(w, (2, 3, 1, 0)).reshape(
            kh, kw * ci, co).astype(jnp.bfloat16)

    w1 = conv_w(res_w1, 3, 3)
    w2 = conv_w(res_w2, 3, 3)
    wd4 = jnp.transpose(down_w, (2, 3, 1, 0)).astype(jnp.bfloat16)
    wde = wd4[:, 1::2].reshape(4, 2 * cout, cout)
    wdo = wd4[:, 0::2].reshape(4, 2 * cout, cout)
    wi0 = id_w.reshape(cout, cin).T.astype(jnp.bfloat16)
    wi = jnp.concatenate(
        [jnp.zeros((cin, cout), jnp.bfloat16), wi0,
         jnp.zeros((cin, cout), jnp.bfloat16)], axis=0)
    b1 = res_b1.reshape(1, cout).astype(jnp.float32)
    b2 = res_b2.reshape(1, cout).astype(jnp.float32)
    bi = id_b.reshape(1, cout).astype(jnp.float32)

    out, down = pl.pallas_call(
        functools.partial(_fused_kernel, H=H, W=W, slope=slope),
        out_shape=[
            jax.ShapeDtypeStruct((N, cout, H * W), jnp.float32),
            jax.ShapeDtypeStruct((N, cout, Ho * Wo), jnp.float32),
        ],
        grid_spec=pltpu.PrefetchScalarGridSpec(
            num_scalar_prefetch=0,
            grid=(N,),
            in_specs=[
                pl.BlockSpec((1, cin, H * W), lambda n: (n, 0, 0)),
                pl.BlockSpec((3, 3 * cin, cout), lambda n: (0, 0, 0)),
                pl.BlockSpec((1, cout), lambda n: (0, 0)),
                pl.BlockSpec((3, 3 * cout, cout), lambda n: (0, 0, 0)),
                pl.BlockSpec((1, cout), lambda n: (0, 0)),
                pl.BlockSpec((3 * cin, cout), lambda n: (0, 0)),
                pl.BlockSpec((1, cout), lambda n: (0, 0)),
                pl.BlockSpec((4, 2 * cout, cout), lambda n: (0, 0, 0)),
                pl.BlockSpec((4, 2 * cout, cout), lambda n: (0, 0, 0)),
            ],
            out_specs=[
                pl.BlockSpec((1, cout, H * W), lambda n: (n, 0, 0)),
                pl.BlockSpec((1, cout, Ho * Wo), lambda n: (n, 0, 0)),
            ],
        ),
        compiler_params=pltpu.CompilerParams(
            dimension_semantics=("parallel",), vmem_limit_bytes=_VMEM_LIMIT),
    )(x.reshape(N, cin, H * W), w1, b1, w2, b2, wi, bi, wde, wdo)

    return down.reshape(N, cout, Ho, Wo), out.reshape(N, cout, H, W)
